# Initial kernel scaffold; baseline (speedup 1.0000x reference)
import functools

import jax
import jax.numpy as jnp
from jax import lax
from jax.experimental import pallas as pl
from jax.experimental.pallas import tpu as pltpu
from jax.experimental.pallas import tpu_sc as plsc

N = 10000
E = 320000
D = 128

NC = 2
NS = 16
NW = NC * NS

K = 128
CPT = -(-E // (NW * K))
CHUNKS = NW * CPT
EP = CHUNKS * K
NP = 10240
RPT = NP // NS
RB = 128
GB = NP // RB

_mesh = plsc.VectorSubcoreMesh(
    core_axis_name="c", subcore_axis_name="s", num_cores=NC, num_subcores=NS
)


@functools.partial(
    pl.kernel,
    out_type=jax.ShapeDtypeStruct((NC, NP, 16), jnp.float32),
    mesh=_mesh,
    scratch_types=[
        pltpu.VMEM((2, K), jnp.int32),
        pltpu.VMEM((K, 16), jnp.float32),
        pltpu.VMEM_SHARED((NP, 16), jnp.float32),
    ],
)
def _sc_degree(sd_hbm, ones_hbm, zeros_hbm, out_hbm, sd_v, ones_v, deg_sp):
    c = lax.axis_index("c")
    s = lax.axis_index("s")
    wid = c * NS + s
    pltpu.sync_copy(ones_hbm, ones_v)
    pltpu.sync_copy(
        zeros_hbm.at[pl.ds(s * RPT, RPT)], deg_sp.at[pl.ds(s * RPT, RPT)]
    )
    plsc.subcore_barrier()

    @pl.loop(0, CPT)
    def _(j):
        chunk = wid * CPT + j
        pltpu.sync_copy(sd_hbm.at[chunk], sd_v)
        pltpu.sync_copy(ones_v, deg_sp.at[sd_v.at[1]], add=True)

    plsc.subcore_barrier()
    pltpu.sync_copy(
        deg_sp.at[pl.ds(s * RPT, RPT)], out_hbm.at[c].at[pl.ds(s * RPT, RPT)]
    )


@functools.partial(
    pl.kernel,
    out_type=jax.ShapeDtypeStruct((NC, NP, D), jnp.float32),
    mesh=_mesh,
    scratch_types=[
        pltpu.VMEM((2, K), jnp.int32),
        pltpu.VMEM((K, D), jnp.float32),
        pltpu.VMEM_SHARED((NP, D), jnp.float32),
        pltpu.SemaphoreType.DMA,
    ],
)
def _sc_aggregate(g_hbm, sd_hbm, zeros_hbm, out_hbm, sd_v, rows_v, acc_sp, sem):
    c = lax.axis_index("c")
    s = lax.axis_index("s")
    wid = c * NS + s
    pltpu.sync_copy(
        zeros_hbm.at[pl.ds(s * RPT, RPT)], acc_sp.at[pl.ds(s * RPT, RPT)]
    )
    plsc.subcore_barrier()

    @pl.loop(0, CPT)
    def _(j):
        chunk = wid * CPT + j
        pltpu.sync_copy(sd_hbm.at[chunk], sd_v)
        pltpu.async_copy(g_hbm.at[sd_v.at[0]], rows_v, sem).wait()
        pltpu.sync_copy(rows_v, acc_sp.at[sd_v.at[1]], add=True)

    plsc.subcore_barrier()
    pltpu.sync_copy(
        acc_sp.at[pl.ds(s * RPT, RPT)], out_hbm.at[c].at[pl.ds(s * RPT, RPT)]
    )


def _dis_block(degp):
    indeg = degp[0, :, 0:1] + degp[1, :, 0:1]
    return lax.rsqrt(1.0 + indeg)


def _pre_body(x_ref, degp_ref, w_ref, out_ref):
    dis = _dis_block(degp_ref[...])
    h = jnp.dot(x_ref[...], w_ref[...], preferred_element_type=jnp.float32)
    out_ref[...] = h * dis


def _mid_body(acc_ref, g_ref, degp_ref, w_ref, b_ref, out_ref):
    dis = _dis_block(degp_ref[...])
    sagg = acc_ref[0] + acc_ref[1] + g_ref[...]
    t = jnp.maximum(dis * sagg + b_ref[...], 0.0)
    out_ref[...] = jnp.dot(t, w_ref[...], preferred_element_type=jnp.float32) * dis


def _out_body(acc_ref, g_ref, degp_ref, wo_ref, b_ref, bo_ref, out_ref):
    dis = _dis_block(degp_ref[...])
    sagg = acc_ref[0] + acc_ref[1] + g_ref[...]
    t = jnp.maximum(dis * sagg + b_ref[...], 0.0)
    z = jnp.dot(t, wo_ref[...], preferred_element_type=jnp.float32) + bo_ref[...]
    out_ref[...] = jax.nn.sigmoid(z)


_row_spec = pl.BlockSpec((RB, D), lambda i: (i, 0))
_degp_spec = pl.BlockSpec((NC, RB, 16), lambda i: (0, i, 0))
_acc_spec = pl.BlockSpec((NC, RB, D), lambda i: (0, i, 0))
_w_spec = pl.BlockSpec((D, D), lambda i: (0, 0))
_b_spec = pl.BlockSpec((1, D), lambda i: (0, 0))

_tc_pre = pl.pallas_call(
    _pre_body,
    grid=(GB,),
    in_specs=[_row_spec, _degp_spec, _w_spec],
    out_specs=_row_spec,
    out_shape=jax.ShapeDtypeStruct((NP, D), jnp.float32),
)

_tc_mid = pl.pallas_call(
    _mid_body,
    grid=(GB,),
    in_specs=[_acc_spec, _row_spec, _degp_spec, _w_spec, _b_spec],
    out_specs=_row_spec,
    out_shape=jax.ShapeDtypeStruct((NP, D), jnp.float32),
)

_tc_out = pl.pallas_call(
    _out_body,
    grid=(GB,),
    in_specs=[
        _acc_spec,
        _row_spec,
        _degp_spec,
        pl.BlockSpec((D, 1), lambda i: (0, 0)),
        _b_spec,
        pl.BlockSpec((1, 1), lambda i: (0, 0)),
    ],
    out_specs=pl.BlockSpec((RB, 1), lambda i: (i, 0)),
    out_shape=jax.ShapeDtypeStruct((NP, 1), jnp.float32),
)


@jax.jit
def kernel(x, edge_index, W1, b1, W2, b2, W3, b3, Wo, bo):
    ei = jnp.pad(edge_index, ((0, 0), (0, EP - E)), constant_values=N)
    sd = ei.reshape(2, CHUNKS, K).transpose(1, 0, 2)
    xp = jnp.pad(x, ((0, NP - N), (0, 0)))
    ones16 = jnp.ones((K, 16), jnp.float32)
    zeros_d = jnp.zeros((NP, D), jnp.float32)
    zeros16 = jnp.zeros((NP, 16), jnp.float32)

    degp = _sc_degree(sd, ones16, zeros16)
    g1 = _tc_pre(xp, degp, W1)
    a1 = _sc_aggregate(g1, sd, zeros_d)
    g2 = _tc_mid(a1, g1, degp, W2, b1.reshape(1, D))
    a2 = _sc_aggregate(g2, sd, zeros_d)
    g3 = _tc_mid(a2, g2, degp, W3, b2.reshape(1, D))
    a3 = _sc_aggregate(g3, sd, zeros_d)
    y = _tc_out(a3, g3, degp, Wo, b3.reshape(1, D), bo.reshape(1, 1))
    return y[:N]


# speedup vs baseline: 6.7259x; 6.7259x over previous
import functools

import jax
import jax.numpy as jnp
from jax import lax
from jax.experimental import pallas as pl
from jax.experimental.pallas import tpu as pltpu
from jax.experimental.pallas import tpu_sc as plsc

N = 10000
E = 320000
D = 128

NC = 2
NS = 16
NW = NC * NS

K = 128
CPT = -(-E // (NW * K))
CHUNKS = NW * CPT
EP = CHUNKS * K
NP = 10240
RPT = NP // NS
RB = 128
GB = NP // RB

_mesh = plsc.VectorSubcoreMesh(
    core_axis_name="c", subcore_axis_name="s", num_cores=NC, num_subcores=NS
)


@functools.partial(
    pl.kernel,
    out_type=jax.ShapeDtypeStruct((NC, NP, D), jnp.float32),
    mesh=_mesh,
    scratch_types=[
        pltpu.VMEM((2, K), jnp.int32),
        pltpu.VMEM((K, D), jnp.float32),
        pltpu.VMEM_SHARED((NP, D), jnp.float32),
        pltpu.SemaphoreType.DMA,
    ],
)
def _sc_aggregate(g_hbm, sd_hbm, zeros_hbm, out_hbm, sd_v, rows_v, acc_sp, sem):
    c = lax.axis_index("c")
    s = lax.axis_index("s")
    wid = c * NS + s
    pltpu.sync_copy(
        zeros_hbm.at[pl.ds(s * RPT, RPT)], acc_sp.at[pl.ds(s * RPT, RPT)]
    )
    plsc.subcore_barrier()

    @pl.loop(0, CPT)
    def _(j):
        chunk = wid * CPT + j
        pltpu.sync_copy(sd_hbm.at[chunk], sd_v)
        pltpu.async_copy(g_hbm.at[sd_v.at[0]], rows_v, sem).wait()
        pltpu.sync_copy(rows_v, acc_sp.at[sd_v.at[1]], add=True)

    plsc.subcore_barrier()
    pltpu.sync_copy(
        acc_sp.at[pl.ds(s * RPT, RPT)], out_hbm.at[c].at[pl.ds(s * RPT, RPT)]
    )


def _dis_block(degp):
    indeg = degp[0, :, 0:1] + degp[1, :, 0:1]
    return lax.rsqrt(1.0 + indeg)


def _pre_body(x_ref, degp_ref, w_ref, out_ref):
    dis = _dis_block(degp_ref[...])
    h = jnp.dot(x_ref[...], w_ref[...], preferred_element_type=jnp.float32)
    out_ref[...] = h * dis


def _mid_body(acc_ref, g_ref, degp_ref, w_ref, b_ref, out_ref):
    dis = _dis_block(degp_ref[...])
    sagg = acc_ref[0] + acc_ref[1] + g_ref[...]
    t = jnp.maximum(dis * sagg + b_ref[...], 0.0)
    out_ref[...] = jnp.dot(t, w_ref[...], preferred_element_type=jnp.float32) * dis


def _out_body(acc_ref, g_ref, degp_ref, wo_ref, b_ref, bo_ref, out_ref):
    dis = _dis_block(degp_ref[...])
    sagg = acc_ref[0] + acc_ref[1] + g_ref[...]
    t = jnp.maximum(dis * sagg + b_ref[...], 0.0)
    z = jnp.dot(t, wo_ref[...], preferred_element_type=jnp.float32) + bo_ref[...]
    out_ref[...] = jax.nn.sigmoid(z)


_row_spec = pl.BlockSpec((RB, D), lambda i: (i, 0))
_acc_spec = pl.BlockSpec((NC, RB, D), lambda i: (0, i, 0))
_degp_spec = _acc_spec
_w_spec = pl.BlockSpec((D, D), lambda i: (0, 0))
_b_spec = pl.BlockSpec((1, D), lambda i: (0, 0))

_tc_pre = pl.pallas_call(
    _pre_body,
    grid=(GB,),
    in_specs=[_row_spec, _degp_spec, _w_spec],
    out_specs=_row_spec,
    out_shape=jax.ShapeDtypeStruct((NP, D), jnp.float32),
)

_tc_mid = pl.pallas_call(
    _mid_body,
    grid=(GB,),
    in_specs=[_acc_spec, _row_spec, _degp_spec, _w_spec, _b_spec],
    out_specs=_row_spec,
    out_shape=jax.ShapeDtypeStruct((NP, D), jnp.float32),
)

_tc_out = pl.pallas_call(
    _out_body,
    grid=(GB,),
    in_specs=[
        _acc_spec,
        _row_spec,
        _degp_spec,
        pl.BlockSpec((D, 1), lambda i: (0, 0)),
        _b_spec,
        pl.BlockSpec((1, 1), lambda i: (0, 0)),
    ],
    out_specs=pl.BlockSpec((RB, 1), lambda i: (i, 0)),
    out_shape=jax.ShapeDtypeStruct((NP, 1), jnp.float32),
)


@jax.jit
def kernel(x, edge_index, W1, b1, W2, b2, W3, b3, Wo, bo):
    ei = jnp.pad(edge_index, ((0, 0), (0, EP - E)), constant_values=N)
    sd = ei.reshape(2, CHUNKS, K).transpose(1, 0, 2)
    xp = jnp.pad(x, ((0, NP - N), (0, 0)))
    ones_d = jnp.ones((NP, D), jnp.float32)
    zeros_d = jnp.zeros((NP, D), jnp.float32)

    degp = _sc_aggregate(ones_d, sd, zeros_d)
    g1 = _tc_pre(xp, degp, W1)
    a1 = _sc_aggregate(g1, sd, zeros_d)
    g2 = _tc_mid(a1, g1, degp, W2, b1.reshape(1, D))
    a2 = _sc_aggregate(g2, sd, zeros_d)
    g3 = _tc_mid(a2, g2, degp, W3, b2.reshape(1, D))
    a3 = _sc_aggregate(g3, sd, zeros_d)
    y = _tc_out(a3, g3, degp, Wo, b3.reshape(1, D), bo.reshape(1, 1))
    return y[:N]
